# baseline (speedup 1.0000x reference)
import jax
import jax.numpy as jnp
from jax import lax
from jax.experimental import pallas as pl
from jax.experimental.pallas import tpu as pltpu
from jax.experimental.pallas import tpu_sc as plsc

_N = 8192
_K = 8192
_D = 256
_NB = 1024
_KB = 1024
_GN = _N // _NB
_GK = _K // _KB


def _argmin_body(z_ref, e_ref, idx_ref, nrm_s, bestv_s, besti_s):
    n = pl.program_id(0)
    k = pl.program_id(1)
    e = e_ref[...]

    @pl.when(n == 0)
    def _():
        ef = e.astype(jnp.float32)
        nrm_s[k, :] = -0.5 * jnp.sum(ef * ef, axis=1)

    scores = lax.dot_general(
        z_ref[...], e, (((1,), (1,)), ((), ())),
        preferred_element_type=jnp.float32)
    val = scores + nrm_s[k, :][None, :]
    lmax = jnp.max(val, axis=1, keepdims=True)
    col = lax.broadcasted_iota(jnp.int32, (_NB, _KB), 1)
    lidx = jnp.min(jnp.where(val == lmax, col, jnp.int32(2**30)),
                   axis=1, keepdims=True) + k * _KB

    @pl.when(k == 0)
    def _():
        bestv_s[...] = lmax
        besti_s[...] = lidx

    @pl.when(k > 0)
    def _():
        better = lmax > bestv_s[...]
        besti_s[...] = jnp.where(better, lidx, besti_s[...])
        bestv_s[...] = jnp.where(better, lmax, bestv_s[...])

    @pl.when(k == _GK - 1)
    def _():
        b = besti_s[...]
        idx_ref[...] = jnp.concatenate([2 * b, 2 * b + 1], axis=1)


_argmin_call = pl.pallas_call(
    _argmin_body,
    grid=(_GN, _GK),
    in_specs=[
        pl.BlockSpec((_NB, _D), lambda n, k: (n, 0)),
        pl.BlockSpec((_KB, _D), lambda n, k: (k, 0)),
    ],
    out_specs=pl.BlockSpec((_NB, 2), lambda n, k: (n, 0)),
    out_shape=jax.ShapeDtypeStruct((_N, 2), jnp.int32),
    scratch_shapes=[
        pltpu.VMEM((_GK, _KB), jnp.float32),
        pltpu.VMEM((_NB, 1), jnp.float32),
        pltpu.VMEM((_NB, 1), jnp.int32),
    ],
    compiler_params=pltpu.CompilerParams(
        dimension_semantics=("arbitrary", "arbitrary")),
)

_W = 128
_NH = _N * 2
_DH = _D // 2


def _gather_combine(emb_half, idx2_row, z_half):
    mesh = plsc.VectorSubcoreMesh(core_axis_name="c", subcore_axis_name="s")

    @pl.kernel(out_type=jax.ShapeDtypeStruct((_NH, _DH), jnp.float32),
               mesh=mesh)
    def sc_kernel(emb_hbm, i_hbm, z_hbm, o_hbm):
        def body(i_vmem, z_vmem, o_vmem):
            pltpu.sync_copy(emb_hbm.at[i_vmem.at[0]], o_vmem)

            @pl.loop(0, _W)
            def _(r):
                for c in range(0, _DH, 16):
                    slc = (pl.ds(r, 1), pl.ds(c, 16))
                    o_vmem.at[slc][...] = (2.0 * o_vmem.at[slc][...]
                                           - z_vmem.at[slc][...])

        pltpu.emit_pipeline(
            body,
            grid=(_NH // _W,),
            in_specs=[pl.BlockSpec((1, _W), lambda i: (0, i)),
                      pl.BlockSpec((_W, _DH), lambda i: (i, 0))],
            out_specs=[pl.BlockSpec((_W, _DH), lambda i: (i, 0))],
            core_axis_name=("c", "s"),
            dimension_semantics=(pltpu.PARALLEL,),
        )(i_hbm, z_hbm, o_hbm)

    return sc_kernel(emb_half, idx2_row, z_half)


def kernel(z, emb):
    z_flat = z.reshape(_N, _D)
    idx2 = _argmin_call(z_flat.astype(jnp.bfloat16), emb.astype(jnp.bfloat16))
    out = _gather_combine(emb.reshape(_NH, _DH), idx2.reshape(1, _NH),
                          z.reshape(_NH, _DH))
    return out.reshape(z.shape)
